# baseline (speedup 1.0000x reference)
import jax
import jax.numpy as jnp
from jax import lax
from jax.experimental import pallas as pl
from jax.experimental.pallas import tpu as pltpu
from jax.experimental.pallas import tpu_sc as plsc

N = 10000
E = 160000
D = 256
ED = 16
HALF = 128
OUT = 128
L = 3
BN_EPS = 1e-5

NC = 2
NS = 16
CH = 64
EP = 163840
EPT = EP // NS
NCH = EPT // CH
RPT = N // NS
RCH = 125
NRC = RPT // RCH


def _edge_agg_body(x2, e2, src, dst, out,
                   is0, is1, id0, id1, x0, x1, e0, e1, ibuf,
                   agg, sg0, sg1, se0, se1, ss0, ss1):
    c = lax.axis_index("c")
    s = lax.axis_index("s")
    cN = c * N

    isb = (is0, is1)
    idb = (id0, id1)
    xb = (x0, x1)
    eb = (e0, e1)
    sg = (sg0, sg1)
    se = (se0, se1)
    ss = (ss0, ss1)

    for j in range(NRC):
        base = s * RPT + j * RCH
        pltpu.sync_copy(x2.at[pl.ds(cN + base, RCH)], ibuf)
        pltpu.sync_copy(ibuf, agg.at[pl.ds(base, RCH)])
    plsc.subcore_barrier()

    def issue(j, b):
        ebase = s * EPT + j * CH
        pltpu.sync_copy(src.at[pl.ds(ebase, CH)], isb[b])
        pltpu.sync_copy(dst.at[pl.ds(ebase, CH)], idb[b])
        for v in range(CH // 16):
            sl = pl.ds(v * 16, 16)
            isb[b][sl] = isb[b][sl] + cN
        pltpu.async_copy(x2.at[isb[b]], xb[b], sg[b])
        pltpu.async_copy(e2.at[pl.ds(c * EP + ebase, CH)], eb[b], se[b])

    def compute(b):
        xr, er = xb[b], eb[b]

        def row(r, rc):
            for v in range(HALF // 16):
                sl = pl.ds(v * 16, 16)
                xr[r, sl] = jnp.maximum(xr[r, sl] + er[r, sl], 0.0)
            return rc

        lax.fori_loop(0, CH, row, 0, unroll=4)

    def wait_loads(b):
        pltpu.make_async_copy(x2.at[isb[b]], xb[b], sg[b]).wait()
        pltpu.make_async_copy(e2.at[pl.ds(c * EP, CH)], eb[b], se[b]).wait()

    def wait_scatter(b):
        pltpu.make_async_copy(xb[b], agg.at[idb[b]], ss[b]).wait()

    def scatter(b):
        pltpu.async_copy(xb[b], agg.at[idb[b]], ss[b], add=True)

    issue(0, 0)
    issue(1, 1)
    wait_loads(0)
    compute(0)
    scatter(0)

    def two_chunks(i, carry):
        j1 = 2 * i + 1
        wait_scatter(0)
        issue(j1 + 1, 0)
        wait_loads(1)
        compute(1)
        scatter(1)
        wait_scatter(1)
        issue(j1 + 2, 1)
        wait_loads(0)
        compute(0)
        scatter(0)
        return carry

    lax.fori_loop(0, NCH // 2 - 1, two_chunks, 0)
    wait_scatter(0)
    wait_loads(1)
    compute(1)
    scatter(1)
    wait_scatter(1)
    plsc.subcore_barrier()

    for j in range(NRC):
        base = s * RPT + j * RCH
        pltpu.sync_copy(agg.at[pl.ds(base, RCH)], ibuf)
        pltpu.sync_copy(ibuf, out.at[pl.ds(cN + base, RCH)])


_EDGE_AGG_CACHE = []


def _edge_agg(h2, e2, src, dst):
    if not _EDGE_AGG_CACHE:
        _EDGE_AGG_CACHE.append(pl.kernel(
            _edge_agg_body,
            out_type=jax.ShapeDtypeStruct((NC * N, HALF), jnp.float32),
            mesh=plsc.VectorSubcoreMesh(core_axis_name="c",
                                        subcore_axis_name="s",
                                        num_cores=NC, num_subcores=NS),
            scratch_types=[
                pltpu.VMEM((CH,), jnp.int32),
                pltpu.VMEM((CH,), jnp.int32),
                pltpu.VMEM((CH,), jnp.int32),
                pltpu.VMEM((CH,), jnp.int32),
                pltpu.VMEM((CH, HALF), jnp.float32),
                pltpu.VMEM((CH, HALF), jnp.float32),
                pltpu.VMEM((CH, HALF), jnp.float32),
                pltpu.VMEM((CH, HALF), jnp.float32),
                pltpu.VMEM((RCH, HALF), jnp.float32),
                pltpu.VMEM_SHARED((N + 8, HALF), jnp.float32),
                pltpu.SemaphoreType.DMA,
                pltpu.SemaphoreType.DMA,
                pltpu.SemaphoreType.DMA,
                pltpu.SemaphoreType.DMA,
                pltpu.SemaphoreType.DMA,
                pltpu.SemaphoreType.DMA,
            ],
            compiler_params=pltpu.CompilerParams(use_tc_tiling_on_sc=False),
        ))
    return _EDGE_AGG_CACHE[0](h2, e2, src, dst)


BE = 2048
RB = 1000


def _eproj_body(ea_ref, w_ref, b_ref, out_ref):
    out_ref[0] = (
        jnp.dot(ea_ref[...], w_ref[0], preferred_element_type=jnp.float32)
        + b_ref[0]
    )


def _eproj(ea, w_split, b_split):
    return pl.pallas_call(
        _eproj_body,
        grid=(NC, EP // BE),
        in_specs=[
            pl.BlockSpec((BE, ED), lambda c, i: (i, 0)),
            pl.BlockSpec((1, ED, HALF), lambda c, i: (c, 0, 0)),
            pl.BlockSpec((1, 1, HALF), lambda c, i: (c, 0, 0)),
        ],
        out_specs=pl.BlockSpec((1, BE, HALF), lambda c, i: (c, i, 0)),
        out_shape=jax.ShapeDtypeStruct((NC, EP, HALF), jnp.float32),
    )(ea, w_split, b_split)


def _mlp_body(xa_ref, w1_ref, b1_ref, sc_ref, bt_ref, w2_ref, b2_ref,
              out_ref):
    h = jnp.concatenate([xa_ref[0], xa_ref[1]], axis=1)
    t = jnp.dot(h, w1_ref[...], preferred_element_type=jnp.float32) + b1_ref[...]
    t = t * sc_ref[...] + bt_ref[...]
    t = jnp.maximum(t, 0.0)
    t = jnp.dot(t, w2_ref[...], preferred_element_type=jnp.float32) + b2_ref[...]
    t = jnp.maximum(t, 0.0)
    out_ref[0] = t[:, :HALF]
    out_ref[1] = t[:, HALF:]


def _mlp(xa, w1, b1r, scr, btr, w2, b2r):
    return pl.pallas_call(
        _mlp_body,
        grid=(N // RB,),
        in_specs=[
            pl.BlockSpec((NC, RB, HALF), lambda i: (0, i, 0)),
            pl.BlockSpec((D, D), lambda i: (0, 0)),
            pl.BlockSpec((1, D), lambda i: (0, 0)),
            pl.BlockSpec((1, D), lambda i: (0, 0)),
            pl.BlockSpec((1, D), lambda i: (0, 0)),
            pl.BlockSpec((D, D), lambda i: (0, 0)),
            pl.BlockSpec((1, D), lambda i: (0, 0)),
        ],
        out_specs=pl.BlockSpec((NC, RB, HALF), lambda i: (0, i, 0)),
        out_shape=jax.ShapeDtypeStruct((NC, N, HALF), jnp.float32),
    )(xa, w1, b1r, scr, btr, w2, b2r)


def _clf_body(xa_ref, w1_ref, b1_ref, w2_ref, b2_ref, out_ref):
    h = jnp.concatenate([xa_ref[0], xa_ref[1]], axis=1)
    t = jnp.dot(h, w1_ref[...], preferred_element_type=jnp.float32) + b1_ref[...]
    t = jnp.maximum(t, 0.0)
    out_ref[...] = (
        jnp.dot(t, w2_ref[...], preferred_element_type=jnp.float32) + b2_ref[...]
    )


def _clf(xa, wc1, bc1r, wc2, bc2r):
    return pl.pallas_call(
        _clf_body,
        grid=(N // RB,),
        in_specs=[
            pl.BlockSpec((NC, RB, HALF), lambda i: (0, i, 0)),
            pl.BlockSpec((D, D), lambda i: (0, 0)),
            pl.BlockSpec((1, D), lambda i: (0, 0)),
            pl.BlockSpec((D, OUT), lambda i: (0, 0)),
            pl.BlockSpec((1, OUT), lambda i: (0, 0)),
        ],
        out_specs=pl.BlockSpec((RB, OUT), lambda i: (i, 0)),
        out_shape=jax.ShapeDtypeStruct((N, OUT), jnp.float32),
    )(xa, wc1, bc1r, wc2, bc2r)


def kernel(x, edge_index, edge_attr, We, be, W1, b1, g1, bt1, W2, b2,
           Wc1, bc1, Wc2, bc2):
    pad = EP - E
    src = jnp.concatenate(
        [edge_index[0].astype(jnp.int32), jnp.zeros((pad,), jnp.int32)])
    dst = jnp.concatenate(
        [edge_index[1].astype(jnp.int32), jnp.full((pad,), N, jnp.int32)])
    ea = jnp.concatenate(
        [edge_attr, jnp.zeros((pad, ED), jnp.float32)], axis=0)
    scale = g1 / jnp.sqrt(1.0 + BN_EPS)

    e2s = []
    for l in range(L):
        w_split = We[l].reshape(ED, NC, HALF).transpose(1, 0, 2)
        b_split = be[l].reshape(NC, 1, HALF)
        e2s.append(_eproj(ea, w_split, b_split).reshape(NC * EP, HALF))

    h2 = x.reshape(N, NC, HALF).transpose(1, 0, 2).reshape(NC * N, HALF)
    for l in range(L):
        xa = _edge_agg(h2, e2s[l], src, dst)
        h2 = _mlp(xa.reshape(NC, N, HALF), W1[l], b1[l][None], scale[l][None],
                  bt1[l][None], W2[l], b2[l][None]).reshape(NC * N, HALF)
    return _clf(h2.reshape(NC, N, HALF), Wc1, bc1[None], Wc2, bc2[None])


# speedup vs baseline: 1.1247x; 1.1247x over previous
import jax
import jax.numpy as jnp
from jax import lax
from jax.experimental import pallas as pl
from jax.experimental.pallas import tpu as pltpu
from jax.experimental.pallas import tpu_sc as plsc

N = 10000
E = 160000
D = 256
ED = 16
HALF = 128
OUT = 128
L = 3
BN_EPS = 1e-5

NC = 2
NS = 16
CH = 80
EP = 163840
EPT = EP // NS
NCH = EPT // CH
IB = 32
NBLK = NCH // IB
RPT = N // NS


def _edge_agg_body(x2, e2, src2, dst2, out,
                   isb, idb, x0, x1, e0, e1,
                   agg, sg0, sg1, se0, se1, ss0, ss1):
    c = lax.axis_index("c")
    s = lax.axis_index("s")
    cN = c * N

    xb = (x0, x1)
    eb = (e0, e1)
    sg = (sg0, sg1)
    se = (se0, se1)
    ss = (ss0, ss1)

    def stage_rows(src_ref, dst_ref, nrows):
        pltpu.sync_copy(src_ref, x0.at[pl.ds(0, nrows)])
        pltpu.sync_copy(x0.at[pl.ds(0, nrows)], dst_ref)

    for j in range(7):
        base = s * RPT + j * CH
        stage_rows(x2.at[pl.ds(cN + base, CH)], agg.at[pl.ds(base, CH)], CH)
    tbase = s * RPT + 7 * CH
    stage_rows(x2.at[pl.ds(cN + tbase, 65)], agg.at[pl.ds(tbase, 65)], 65)
    plsc.subcore_barrier()

    def refill(blk):
        row0 = s * NCH + blk * IB
        pltpu.sync_copy(src2.at[pl.ds(row0, IB)], isb)
        pltpu.sync_copy(dst2.at[pl.ds(row0, IB)], idb)

        def off_row(k, carry):
            for v in range(CH // 16):
                sl = pl.ds(v * 16, 16)
                isb[k, sl] = isb[k, sl] + cN
            return carry

        lax.fori_loop(0, IB, off_row, 0, unroll=4)

    def issue(blk, k, b):
        ebase = c * EP + s * EPT + (blk * IB + k) * CH
        pltpu.async_copy(x2.at[isb.at[k]], xb[b], sg[b])
        pltpu.async_copy(e2.at[pl.ds(ebase, CH)], eb[b], se[b])

    def compute(b):
        xr, er = xb[b], eb[b]

        def row(r, rc):
            for v in range(HALF // 16):
                sl = pl.ds(v * 16, 16)
                xr[r, sl] = jnp.maximum(xr[r, sl] + er[r, sl], 0.0)
            return rc

        lax.fori_loop(0, CH, row, 0, unroll=4)

    def wait_loads(b):
        pltpu.make_async_copy(x2.at[isb.at[0]], xb[b], sg[b]).wait()
        pltpu.make_async_copy(e2.at[pl.ds(c * EP, CH)], eb[b], se[b]).wait()

    def scatter(k, b):
        pltpu.async_copy(xb[b], agg.at[idb.at[k]], ss[b], add=True)

    def wait_scatter(b):
        pltpu.make_async_copy(xb[b], agg.at[idb.at[0]], ss[b]).wait()

    for blk in range(NBLK):
        refill(blk)
        issue(blk, 0, 0)
        issue(blk, 1, 1)
        wait_loads(0)
        compute(0)
        scatter(0, 0)

        def two_chunks(i, carry):
            k1 = 2 * i + 1
            wait_scatter(0)
            issue(blk, k1 + 1, 0)
            wait_loads(1)
            compute(1)
            scatter(k1, 1)
            wait_scatter(1)
            issue(blk, k1 + 2, 1)
            wait_loads(0)
            compute(0)
            scatter(k1 + 1, 0)
            return carry

        lax.fori_loop(0, IB // 2 - 1, two_chunks, 0)
        wait_scatter(0)
        wait_loads(1)
        compute(1)
        scatter(IB - 1, 1)
        wait_scatter(1)

    plsc.subcore_barrier()
    for j in range(7):
        base = s * RPT + j * CH
        stage_rows(agg.at[pl.ds(base, CH)], out.at[pl.ds(cN + base, CH)], CH)
    tbase = s * RPT + 7 * CH
    stage_rows(agg.at[pl.ds(tbase, 65)], out.at[pl.ds(cN + tbase, 65)], 65)


_EDGE_AGG_CACHE = []


def _edge_agg(h2, e2, src, dst):
    if not _EDGE_AGG_CACHE:
        _EDGE_AGG_CACHE.append(pl.kernel(
            _edge_agg_body,
            out_type=jax.ShapeDtypeStruct((NC * N, HALF), jnp.float32),
            mesh=plsc.VectorSubcoreMesh(core_axis_name="c",
                                        subcore_axis_name="s",
                                        num_cores=NC, num_subcores=NS),
            scratch_types=[
                pltpu.VMEM((IB, CH), jnp.int32),
                pltpu.VMEM((IB, CH), jnp.int32),
                pltpu.VMEM((CH, HALF), jnp.float32),
                pltpu.VMEM((CH, HALF), jnp.float32),
                pltpu.VMEM((CH, HALF), jnp.float32),
                pltpu.VMEM((CH, HALF), jnp.float32),
                pltpu.VMEM_SHARED((N + 8, HALF), jnp.float32),
                pltpu.SemaphoreType.DMA,
                pltpu.SemaphoreType.DMA,
                pltpu.SemaphoreType.DMA,
                pltpu.SemaphoreType.DMA,
                pltpu.SemaphoreType.DMA,
                pltpu.SemaphoreType.DMA,
            ],
            compiler_params=pltpu.CompilerParams(use_tc_tiling_on_sc=False),
        ))
    return _EDGE_AGG_CACHE[0](h2, e2, src, dst)


BE = 2048
RB = 1000


def _eproj_body(ea_ref, w_ref, b_ref, out_ref):
    out_ref[0] = (
        jnp.dot(ea_ref[...], w_ref[0], preferred_element_type=jnp.float32)
        + b_ref[0]
    )


def _eproj(ea, w_split, b_split):
    return pl.pallas_call(
        _eproj_body,
        grid=(NC, EP // BE),
        in_specs=[
            pl.BlockSpec((BE, ED), lambda c, i: (i, 0)),
            pl.BlockSpec((1, ED, HALF), lambda c, i: (c, 0, 0)),
            pl.BlockSpec((1, 1, HALF), lambda c, i: (c, 0, 0)),
        ],
        out_specs=pl.BlockSpec((1, BE, HALF), lambda c, i: (c, i, 0)),
        out_shape=jax.ShapeDtypeStruct((NC, EP, HALF), jnp.float32),
    )(ea, w_split, b_split)


def _mlp_body(xa_ref, w1_ref, b1_ref, sc_ref, bt_ref, w2_ref, b2_ref,
              out_ref):
    h = jnp.concatenate([xa_ref[0], xa_ref[1]], axis=1)
    t = jnp.dot(h, w1_ref[...], preferred_element_type=jnp.float32) + b1_ref[...]
    t = t * sc_ref[...] + bt_ref[...]
    t = jnp.maximum(t, 0.0)
    t = jnp.dot(t, w2_ref[...], preferred_element_type=jnp.float32) + b2_ref[...]
    t = jnp.maximum(t, 0.0)
    out_ref[0] = t[:, :HALF]
    out_ref[1] = t[:, HALF:]


def _mlp(xa, w1, b1r, scr, btr, w2, b2r):
    return pl.pallas_call(
        _mlp_body,
        grid=(N // RB,),
        in_specs=[
            pl.BlockSpec((NC, RB, HALF), lambda i: (0, i, 0)),
            pl.BlockSpec((D, D), lambda i: (0, 0)),
            pl.BlockSpec((1, D), lambda i: (0, 0)),
            pl.BlockSpec((1, D), lambda i: (0, 0)),
            pl.BlockSpec((1, D), lambda i: (0, 0)),
            pl.BlockSpec((D, D), lambda i: (0, 0)),
            pl.BlockSpec((1, D), lambda i: (0, 0)),
        ],
        out_specs=pl.BlockSpec((NC, RB, HALF), lambda i: (0, i, 0)),
        out_shape=jax.ShapeDtypeStruct((NC, N, HALF), jnp.float32),
    )(xa, w1, b1r, scr, btr, w2, b2r)


def _clf_body(xa_ref, w1_ref, b1_ref, w2_ref, b2_ref, out_ref):
    h = jnp.concatenate([xa_ref[0], xa_ref[1]], axis=1)
    t = jnp.dot(h, w1_ref[...], preferred_element_type=jnp.float32) + b1_ref[...]
    t = jnp.maximum(t, 0.0)
    out_ref[...] = (
        jnp.dot(t, w2_ref[...], preferred_element_type=jnp.float32) + b2_ref[...]
    )


def _clf(xa, wc1, bc1r, wc2, bc2r):
    return pl.pallas_call(
        _clf_body,
        grid=(N // RB,),
        in_specs=[
            pl.BlockSpec((NC, RB, HALF), lambda i: (0, i, 0)),
            pl.BlockSpec((D, D), lambda i: (0, 0)),
            pl.BlockSpec((1, D), lambda i: (0, 0)),
            pl.BlockSpec((D, OUT), lambda i: (0, 0)),
            pl.BlockSpec((1, OUT), lambda i: (0, 0)),
        ],
        out_specs=pl.BlockSpec((RB, OUT), lambda i: (i, 0)),
        out_shape=jax.ShapeDtypeStruct((N, OUT), jnp.float32),
    )(xa, wc1, bc1r, wc2, bc2r)


def kernel(x, edge_index, edge_attr, We, be, W1, b1, g1, bt1, W2, b2,
           Wc1, bc1, Wc2, bc2):
    pad = EP - E
    src = jnp.concatenate(
        [edge_index[0].astype(jnp.int32), jnp.zeros((pad,), jnp.int32)]
    ).reshape(EP // CH, CH)
    dst = jnp.concatenate(
        [edge_index[1].astype(jnp.int32), jnp.full((pad,), N, jnp.int32)]
    ).reshape(EP // CH, CH)
    ea = jnp.concatenate(
        [edge_attr, jnp.zeros((pad, ED), jnp.float32)], axis=0)
    scale = g1 / jnp.sqrt(1.0 + BN_EPS)

    e2s = []
    for l in range(L):
        w_split = We[l].reshape(ED, NC, HALF).transpose(1, 0, 2)
        b_split = be[l].reshape(NC, 1, HALF)
        e2s.append(_eproj(ea, w_split, b_split).reshape(NC * EP, HALF))

    h2 = x.reshape(N, NC, HALF).transpose(1, 0, 2).reshape(NC * N, HALF)
    for l in range(L):
        xa = _edge_agg(h2, e2s[l], src, dst)
        h2 = _mlp(xa.reshape(NC, N, HALF), W1[l], b1[l][None], scale[l][None],
                  bt1[l][None], W2[l], b2[l][None]).reshape(NC * N, HALF)
    return _clf(h2.reshape(NC, N, HALF), Wc1, bc1[None], Wc2, bc2[None])
